# Initial kernel scaffold; baseline (speedup 1.0000x reference)
import jax
import jax.numpy as jnp
from jax.experimental import pallas as pl


def kernel(a_hat, x, w1, b1, w2, b2, w3, b3):
    raise NotImplementedError("write your pallas kernel here")


# speedup vs baseline: 24.3334x; 24.3334x over previous
import functools

import jax
import jax.numpy as jnp
from jax.experimental import pallas as pl
from jax.experimental.pallas import tpu as pltpu


def _gcn3_kernel(a_ref, x_ref, w1_ref, w2_ref, w3_ref, b1_ref, b2_ref, b3_ref,
                 o_ref, a_bf_ref, z_ref, h1_ref, h2_ref, *, tm, hid1, hid2,
                 out_dim):
    l = pl.program_id(0)
    i = pl.program_id(1)
    f32 = jnp.float32
    bf = jnp.bfloat16

    @pl.when((l == 0) & (i == 0))
    def _():
        z_ref[:, :hid1] = jnp.dot(
            x_ref[...], w1_ref[...], preferred_element_type=f32).astype(bf)

    @pl.when((l == 1) & (i == 0))
    def _():
        z_ref[:, :hid2] = jnp.dot(
            h1_ref[...], w2_ref[...], preferred_element_type=f32).astype(bf)

    @pl.when((l == 2) & (i == 0))
    def _():
        z_ref[:, :out_dim] = jnp.dot(
            h2_ref[...], w3_ref[...], preferred_element_type=f32).astype(bf)

    rows = pl.ds(i * tm, tm)

    @pl.when(l == 0)
    def _():
        a_bf = a_ref[...].astype(bf)
        a_bf_ref[rows, :] = a_bf
        acc = jnp.dot(a_bf, z_ref[:, :hid1], preferred_element_type=f32)
        h1_ref[rows, :] = jnp.maximum(acc + b1_ref[...], 0.0).astype(bf)

    @pl.when(l == 1)
    def _():
        acc = jnp.dot(a_bf_ref[rows, :], z_ref[:, :hid2],
                      preferred_element_type=f32)
        h2_ref[rows, :] = jnp.maximum(acc + b2_ref[...], 0.0).astype(bf)

    @pl.when(l == 2)
    def _():
        acc = jnp.dot(a_bf_ref[rows, :], z_ref[:, :out_dim],
                      preferred_element_type=f32)
        o_ref[rows, :] = acc + b3_ref[...]


def kernel(a_hat, x, w1, b1, w2, b2, w3, b3):
    n = a_hat.shape[0]
    in_dim = x.shape[1]
    hid1 = w1.shape[1]
    hid2 = w2.shape[1]
    out_dim = w3.shape[1]

    tm = min(256, n)
    n_blocks = n // tm
    zw = max(hid1, hid2, out_dim)
    bf = jnp.bfloat16

    body = functools.partial(_gcn3_kernel, tm=tm, hid1=hid1, hid2=hid2,
                             out_dim=out_dim)

    return pl.pallas_call(
        body,
        out_shape=jax.ShapeDtypeStruct((n, out_dim), jnp.float32),
        grid=(3, n_blocks),
        in_specs=[
            pl.BlockSpec((tm, n),
                         lambda l, i: (jnp.where(l == 0, i, n_blocks - 1), 0)),
            pl.BlockSpec((n, in_dim), lambda l, i: (0, 0)),
            pl.BlockSpec((in_dim, hid1), lambda l, i: (0, 0)),
            pl.BlockSpec((hid1, hid2), lambda l, i: (0, 0)),
            pl.BlockSpec((hid2, out_dim), lambda l, i: (0, 0)),
            pl.BlockSpec((1, hid1), lambda l, i: (0, 0)),
            pl.BlockSpec((1, hid2), lambda l, i: (0, 0)),
            pl.BlockSpec((1, out_dim), lambda l, i: (0, 0)),
        ],
        out_specs=pl.BlockSpec((n, out_dim), lambda l, i: (0, 0)),
        scratch_shapes=[
            pltpu.VMEM((n, n), bf),
            pltpu.VMEM((n, zw), bf),
            pltpu.VMEM((n, hid1), bf),
            pltpu.VMEM((n, hid2), bf),
        ],
        compiler_params=pltpu.CompilerParams(
            dimension_semantics=("arbitrary", "arbitrary"),
            vmem_limit_bytes=60 << 20,
        ),
    )(a_hat, x.astype(bf), w1.astype(bf), w2.astype(bf), w3.astype(bf),
      b1.reshape(1, -1), b2.reshape(1, -1), b3.reshape(1, -1))


# speedup vs baseline: 24.6905x; 1.0147x over previous
import functools

import jax
import jax.numpy as jnp
from jax.experimental import pallas as pl
from jax.experimental.pallas import tpu as pltpu


def _gcn3_kernel(a_ref, x_ref, w1_ref, w2_ref, w3_ref, b1_ref, b2_ref, b3_ref,
                 o_ref, a_bf_ref, z_ref, h1_ref, h2_ref, *, tm, out_dim):
    l = pl.program_id(0)
    i = pl.program_id(1)
    f32 = jnp.float32
    bf = jnp.bfloat16

    @pl.when((l == 0) & (i == 0))
    def _():
        z_ref[...] = jnp.dot(
            x_ref[...].astype(bf), w1_ref[...],
            preferred_element_type=f32).astype(bf)

    @pl.when((l == 1) & (i == 0))
    def _():
        z_ref[...] = jnp.dot(
            h1_ref[...], w2_ref[...], preferred_element_type=f32).astype(bf)

    @pl.when((l == 2) & (i == 0))
    def _():
        z_ref[...] = jnp.dot(
            h2_ref[...], w3_ref[...], preferred_element_type=f32).astype(bf)

    rows = pl.ds(i * tm, tm)

    @pl.when(l == 0)
    def _():
        a_bf = a_ref[...].astype(bf)
        a_bf_ref[rows, :] = a_bf
        acc = jnp.dot(a_bf, z_ref[...], preferred_element_type=f32)
        h1_ref[rows, :] = jnp.maximum(acc + b1_ref[...], 0.0).astype(bf)

    @pl.when(l == 1)
    def _():
        acc = jnp.dot(a_bf_ref[rows, :], z_ref[...],
                      preferred_element_type=f32)
        h2_ref[rows, :] = jnp.maximum(acc + b2_ref[...], 0.0).astype(bf)

    @pl.when(l == 2)
    def _():
        acc = jnp.dot(a_bf_ref[rows, :], z_ref[...],
                      preferred_element_type=f32)
        o_ref[rows, :] = acc[:, :out_dim] + b3_ref[...]


def kernel(a_hat, x, w1, b1, w2, b2, w3, b3):
    n = a_hat.shape[0]
    in_dim = x.shape[1]
    hid1 = w1.shape[1]
    hid2 = w2.shape[1]
    out_dim = w3.shape[1]

    tm = min(256, n)
    n_blocks = n // tm
    zw = max(hid1, hid2, out_dim)
    bf = jnp.bfloat16

    def padw(w):
        return jnp.pad(w.astype(bf), ((0, zw - w.shape[0]),
                                      (0, zw - w.shape[1])))

    w1p = padw(w1) if (w1.shape[0] < zw or hid1 < zw) else w1.astype(bf)
    w2p = padw(w2)
    w3p = padw(w3)
    b1p = jnp.pad(b1.reshape(1, -1), ((0, 0), (0, zw - hid1)))
    b2p = jnp.pad(b2.reshape(1, -1), ((0, 0), (0, zw - hid2)))

    body = functools.partial(_gcn3_kernel, tm=tm, out_dim=out_dim)

    return pl.pallas_call(
        body,
        out_shape=jax.ShapeDtypeStruct((n, out_dim), jnp.float32),
        grid=(3, n_blocks),
        in_specs=[
            pl.BlockSpec((tm, n),
                         lambda l, i: (jnp.where(l == 0, i, n_blocks - 1), 0)),
            pl.BlockSpec((n, in_dim), lambda l, i: (0, 0)),
            pl.BlockSpec((in_dim, zw), lambda l, i: (0, 0)),
            pl.BlockSpec((zw, zw), lambda l, i: (0, 0)),
            pl.BlockSpec((zw, zw), lambda l, i: (0, 0)),
            pl.BlockSpec((1, zw), lambda l, i: (0, 0)),
            pl.BlockSpec((1, zw), lambda l, i: (0, 0)),
            pl.BlockSpec((1, out_dim), lambda l, i: (0, 0)),
        ],
        out_specs=pl.BlockSpec((n, out_dim), lambda l, i: (0, 0)),
        scratch_shapes=[
            pltpu.VMEM((n, n), bf),
            pltpu.VMEM((n, zw), bf),
            pltpu.VMEM((n, zw), bf),
            pltpu.VMEM((n, zw), bf),
        ],
        compiler_params=pltpu.CompilerParams(
            dimension_semantics=("arbitrary", "arbitrary"),
            vmem_limit_bytes=60 << 20,
        ),
    )(a_hat, x, w1p, w2p, w3p, b1p, b2p, b3.reshape(1, -1))
